# Initial kernel scaffold; baseline (speedup 1.0000x reference)
import jax
import jax.numpy as jnp
from jax.experimental import pallas as pl


def kernel(data_cat, table_day, table_time):
    raise NotImplementedError("write your pallas kernel here")


# speedup vs baseline: 2.3519x; 2.3519x over previous
import functools

import jax
import jax.numpy as jnp
from jax import lax
from jax.experimental import pallas as pl
from jax.experimental.pallas import tpu as pltpu
from jax.experimental.pallas import tpu_sc as plsc

NUM_TIME = 288
TIME_SIZE = 64
DAY_SIZE = 32
NUM_DAY = 7
OUT_SIZE = TIME_SIZE + DAY_SIZE
NUM_FUSED = NUM_TIME * NUM_DAY

NC = 2
NS = 16
NW = NC * NS
LANES = 16

CHUNK = 512
GROUPS = CHUNK // LANES
GROW = 128
NGATHER = CHUNK // GROW


def _sc_embed(n_tokens):
  per_w = n_tokens // NW
  iters = per_w // CHUNK
  assert per_w % CHUNK == 0

  mesh = plsc.VectorSubcoreMesh(core_axis_name="c", subcore_axis_name="s")

  @functools.partial(
      pl.kernel,
      mesh=mesh,
      out_type=jax.ShapeDtypeStruct((n_tokens, OUT_SIZE), jnp.float32),
      compiler_params=pltpu.CompilerParams(
          needs_layout_passes=False, use_tc_tiling_on_sc=False
      ),
      scratch_types=[
          pltpu.VMEM((2 * CHUNK,), jnp.int32),
          pltpu.VMEM((NGATHER, GROW), jnp.int32),
          pltpu.VMEM((CHUNK, OUT_SIZE), jnp.float32),
          pltpu.SemaphoreType.DMA,
      ],
  )
  def k(idx_hbm, fused_hbm, out_hbm, idx_v, p_v, blk_v, sem):
    wid = lax.axis_index("s") * NC + lax.axis_index("c")
    base = wid * per_w

    @pl.loop(0, iters)
    def _(i):
      tok0 = base + i * CHUNK
      pltpu.sync_copy(idx_hbm.at[pl.ds(2 * tok0, 2 * CHUNK)], idx_v)
      for g in range(GROUPS):
        ii = lax.iota(jnp.int32, 16) * 2 + (32 * g)
        d = plsc.load_gather(idx_v, [ii])
        t = plsc.load_gather(idx_v, [ii + 1])
        d = jnp.clip(d, 0, NUM_DAY - 1)
        t = jnp.clip(t, 0, NUM_TIME - 1)
        p = t * NUM_DAY + d
        p_v[g // 8, pl.ds((g % 8) * 16, 16)] = p
      for j in range(NGATHER):
        pltpu.async_copy(
            fused_hbm.at[p_v.at[j]], blk_v.at[pl.ds(j * GROW, GROW)], sem
        ).wait()
      pltpu.sync_copy(blk_v, out_hbm.at[pl.ds(tok0, CHUNK)])

  return k


def kernel(data_cat, table_day, table_time):
  B, T, _ = data_cat.shape
  n = B * T
  idx_flat = data_cat.astype(jnp.int32).reshape(2 * n)
  f_time = jnp.broadcast_to(
      table_time[:, None, :], (NUM_TIME, NUM_DAY, TIME_SIZE)
  ).reshape(NUM_FUSED, TIME_SIZE)
  f_day = jnp.broadcast_to(
      table_day[None, :, :], (NUM_TIME, NUM_DAY, DAY_SIZE)
  ).reshape(NUM_FUSED, DAY_SIZE)
  fused = jnp.concatenate([f_time, f_day], axis=1)
  out = _sc_embed(n)(idx_flat, fused)
  return out.reshape(B, T, OUT_SIZE)


# speedup vs baseline: 4.2223x; 1.7953x over previous
import functools

import jax
import jax.numpy as jnp
from jax import lax
from jax.experimental import pallas as pl
from jax.experimental.pallas import tpu as pltpu
from jax.experimental.pallas import tpu_sc as plsc

NUM_TIME = 288
TIME_SIZE = 64
DAY_SIZE = 32
NUM_DAY = 7
OUT_SIZE = TIME_SIZE + DAY_SIZE
NUM_FUSED = NUM_TIME * NUM_DAY
FPAD = 128

NC = 2
NS = 16
NW = NC * NS
LANES = 16

CHUNK = 512
GROUPS = CHUNK // LANES
GROW = 128
NGATHER = CHUNK // GROW


def _sc_embed(n_tokens):
  per_w = n_tokens // NW
  iters = per_w // CHUNK
  assert per_w % CHUNK == 0

  mesh = plsc.VectorSubcoreMesh(core_axis_name="c", subcore_axis_name="s")

  @functools.partial(
      pl.kernel,
      mesh=mesh,
      out_type=jax.ShapeDtypeStruct((n_tokens, FPAD), jnp.float32),
      compiler_params=pltpu.CompilerParams(
          needs_layout_passes=False, use_tc_tiling_on_sc=True
      ),
      scratch_types=[
          pltpu.VMEM((CHUNK,), jnp.int32),
          pltpu.VMEM((CHUNK,), jnp.int32),
          pltpu.VMEM((NGATHER, GROW), jnp.int32),
          pltpu.VMEM((CHUNK, FPAD), jnp.float32),
          pltpu.SemaphoreType.DMA,
      ],
  )
  def k(day_hbm, time_hbm, fused_hbm, out_hbm, d_v, t_v, p_v, blk_v, sem):
    wid = lax.axis_index("s") * NC + lax.axis_index("c")
    base = wid * per_w

    @pl.loop(0, iters)
    def _(i):
      tok0 = base + i * CHUNK
      pltpu.sync_copy(day_hbm.at[pl.ds(tok0, CHUNK)], d_v)
      pltpu.sync_copy(time_hbm.at[pl.ds(tok0, CHUNK)], t_v)
      for g in range(GROUPS):
        d = d_v[pl.ds(g * LANES, LANES)]
        t = t_v[pl.ds(g * LANES, LANES)]
        d = jnp.clip(d, 0, NUM_DAY - 1)
        t = jnp.clip(t, 0, NUM_TIME - 1)
        p = t * NUM_DAY + d
        p_v[g // 8, pl.ds((g % 8) * 16, 16)] = p
      for j in range(NGATHER):
        pltpu.async_copy(
            fused_hbm.at[p_v.at[j]], blk_v.at[pl.ds(j * GROW, GROW)], sem
        ).wait()
      pltpu.sync_copy(blk_v, out_hbm.at[pl.ds(tok0, CHUNK)])

  return k


def kernel(data_cat, table_day, table_time):
  B, T, _ = data_cat.shape
  n = B * T
  data_cat = data_cat.astype(jnp.int32)
  day_idx = data_cat[:, :, 0].reshape(n)
  time_idx = data_cat[:, :, 1].reshape(n)
  f_time = jnp.broadcast_to(
      table_time[:, None, :], (NUM_TIME, NUM_DAY, TIME_SIZE)
  ).reshape(NUM_FUSED, TIME_SIZE)
  f_day = jnp.broadcast_to(
      table_day[None, :, :], (NUM_TIME, NUM_DAY, DAY_SIZE)
  ).reshape(NUM_FUSED, DAY_SIZE)
  f_pad = jnp.zeros((NUM_FUSED, FPAD - OUT_SIZE), jnp.float32)
  fused = jnp.concatenate([f_time, f_day, f_pad], axis=1)
  out = _sc_embed(n)(day_idx, time_idx, fused)
  return out[:, :OUT_SIZE].reshape(B, T, OUT_SIZE)


# speedup vs baseline: 4.3396x; 1.0278x over previous
import functools

import jax
import jax.numpy as jnp
from jax import lax
from jax.experimental import pallas as pl
from jax.experimental.pallas import tpu as pltpu
from jax.experimental.pallas import tpu_sc as plsc

NUM_TIME = 288
TIME_SIZE = 64
DAY_SIZE = 32
NUM_DAY = 7
OUT_SIZE = TIME_SIZE + DAY_SIZE
NUM_FUSED = NUM_TIME * NUM_DAY
FPAD = 128

NC = 2
NS = 16
NW = NC * NS
LANES = 16

CHUNK = 256
GROUPS = CHUNK // LANES
GROW = 128
NGATHER = CHUNK // GROW


def _sc_embed(n_tokens):
  per_w = n_tokens // NW
  iters = per_w // CHUNK
  assert per_w % CHUNK == 0 and iters % 2 == 0

  mesh = plsc.VectorSubcoreMesh(core_axis_name="c", subcore_axis_name="s")

  @functools.partial(
      pl.kernel,
      mesh=mesh,
      out_type=jax.ShapeDtypeStruct((n_tokens, FPAD), jnp.float32),
      compiler_params=pltpu.CompilerParams(
          needs_layout_passes=False, use_tc_tiling_on_sc=True
      ),
      scratch_types=[
          pltpu.VMEM((2, CHUNK), jnp.int32),
          pltpu.VMEM((2, CHUNK), jnp.int32),
          pltpu.VMEM((2, NGATHER, GROW), jnp.int32),
          pltpu.VMEM((CHUNK, FPAD), jnp.float32),
          pltpu.VMEM((CHUNK, FPAD), jnp.float32),
          pltpu.SemaphoreType.DMA,
          pltpu.SemaphoreType.DMA,
          pltpu.SemaphoreType.DMA,
          pltpu.SemaphoreType.DMA,
          pltpu.SemaphoreType.DMA,
          pltpu.SemaphoreType.DMA,
      ],
  )
  def k(day_hbm, time_hbm, fused_hbm, out_hbm,
        d_v, t_v, p_v, blk0, blk1, is0, is1, gs0, gs1, ws0, ws1):
    wid = lax.axis_index("s") * NC + lax.axis_index("c")
    base = wid * per_w
    blk = (blk0, blk1)
    isem = (is0, is1)
    gsem = (gs0, gs1)
    wsem = (ws0, ws1)

    def tok0(i):
      return base + i * CHUNK

    def fire_idx(i, b):
      pltpu.async_copy(day_hbm.at[pl.ds(tok0(i), CHUNK)], d_v.at[b], isem[b])
      pltpu.async_copy(time_hbm.at[pl.ds(tok0(i), CHUNK)], t_v.at[b], isem[b])

    def wait_idx(i, b):
      pltpu.make_async_copy(
          day_hbm.at[pl.ds(tok0(i), CHUNK)], d_v.at[b], isem[b]).wait()
      pltpu.make_async_copy(
          time_hbm.at[pl.ds(tok0(i), CHUNK)], t_v.at[b], isem[b]).wait()

    def compute_p(b):
      for g in range(GROUPS):
        d = d_v[b, pl.ds(g * LANES, LANES)]
        t = t_v[b, pl.ds(g * LANES, LANES)]
        d = jnp.clip(d, 0, NUM_DAY - 1)
        t = jnp.clip(t, 0, NUM_TIME - 1)
        p = t * NUM_DAY + d
        p_v[b, g // 8, pl.ds((g % 8) * LANES, LANES)] = p

    def fire_gathers(b):
      for j in range(NGATHER):
        pltpu.async_copy(
            fused_hbm.at[p_v.at[b, j]],
            blk[b].at[pl.ds(j * GROW, GROW)], gsem[b])

    def wait_gathers(b):
      for j in range(NGATHER):
        pltpu.make_async_copy(
            fused_hbm.at[p_v.at[b, j]],
            blk[b].at[pl.ds(j * GROW, GROW)], gsem[b]).wait()

    def fire_wb(i, b):
      pltpu.async_copy(blk[b], out_hbm.at[pl.ds(tok0(i), CHUNK)], wsem[b])

    def wait_wb(i, b):
      pltpu.make_async_copy(
          blk[b], out_hbm.at[pl.ds(tok0(i), CHUNK)], wsem[b]).wait()

    fire_idx(0, 0)
    wait_idx(0, 0)
    compute_p(0)
    fire_idx(1, 1)
    fire_gathers(0)

    @pl.loop(0, iters, step=2)
    def _(o):
      for b in (0, 1):
        i = o + b
        nb = 1 - b
        wait_gathers(b)
        fire_wb(i, b)

        @pl.when(i + 1 < iters)
        def _():
          wait_idx(i + 1, nb)
          compute_p(nb)

          @pl.when(i + 2 < iters)
          def _():
            fire_idx(i + 2, b)

          @pl.when(i > 0)
          def _():
            wait_wb(i - 1, nb)
          fire_gathers(nb)

        @pl.when(jnp.logical_and(i + 1 >= iters, i > 0))
        def _():
          wait_wb(i - 1, nb)

    wait_wb(iters - 1, (iters - 1) % 2)

  return k


def kernel(data_cat, table_day, table_time):
  B, T, _ = data_cat.shape
  n = B * T
  data_cat = data_cat.astype(jnp.int32)
  day_idx = data_cat[:, :, 0].reshape(n)
  time_idx = data_cat[:, :, 1].reshape(n)
  f_time = jnp.broadcast_to(
      table_time[:, None, :], (NUM_TIME, NUM_DAY, TIME_SIZE)
  ).reshape(NUM_FUSED, TIME_SIZE)
  f_day = jnp.broadcast_to(
      table_day[None, :, :], (NUM_TIME, NUM_DAY, DAY_SIZE)
  ).reshape(NUM_FUSED, DAY_SIZE)
  f_pad = jnp.zeros((NUM_FUSED, FPAD - OUT_SIZE), jnp.float32)
  fused = jnp.concatenate([f_time, f_day, f_pad], axis=1)
  out = _sc_embed(n)(day_idx, time_idx, fused)
  return out[:, :OUT_SIZE].reshape(B, T, OUT_SIZE)


# speedup vs baseline: 4.3414x; 1.0004x over previous
import functools

import jax
import jax.numpy as jnp
from jax import lax
from jax.experimental import pallas as pl
from jax.experimental.pallas import tpu as pltpu
from jax.experimental.pallas import tpu_sc as plsc

NUM_TIME = 288
TIME_SIZE = 64
DAY_SIZE = 32
NUM_DAY = 7
OUT_SIZE = TIME_SIZE + DAY_SIZE
NUM_FUSED = NUM_TIME * NUM_DAY
FPAD = 128

NC = 2
NS = 16
NW = NC * NS
LANES = 16

CHUNK = 256
GROUPS = CHUNK // LANES
GROW = 32
NGATHER = CHUNK // GROW


def _sc_embed(n_tokens):
  per_w = n_tokens // NW
  iters = per_w // CHUNK
  assert per_w % CHUNK == 0 and iters % 2 == 0

  mesh = plsc.VectorSubcoreMesh(core_axis_name="c", subcore_axis_name="s")

  @functools.partial(
      pl.kernel,
      mesh=mesh,
      out_type=jax.ShapeDtypeStruct((n_tokens, FPAD), jnp.float32),
      compiler_params=pltpu.CompilerParams(
          needs_layout_passes=False, use_tc_tiling_on_sc=True
      ),
      scratch_types=[
          pltpu.VMEM((2, CHUNK), jnp.int32),
          pltpu.VMEM((2, CHUNK), jnp.int32),
          pltpu.VMEM((2, NGATHER, GROW), jnp.int32),
          pltpu.VMEM((CHUNK, FPAD), jnp.float32),
          pltpu.VMEM((CHUNK, FPAD), jnp.float32),
          pltpu.SemaphoreType.DMA,
          pltpu.SemaphoreType.DMA,
          pltpu.SemaphoreType.DMA,
          pltpu.SemaphoreType.DMA,
          pltpu.SemaphoreType.DMA,
          pltpu.SemaphoreType.DMA,
      ],
  )
  def k(day_hbm, time_hbm, fused_hbm, out_hbm,
        d_v, t_v, p_v, blk0, blk1, is0, is1, gs0, gs1, ws0, ws1):
    wid = lax.axis_index("s") * NC + lax.axis_index("c")
    base = wid * per_w
    blk = (blk0, blk1)
    isem = (is0, is1)
    gsem = (gs0, gs1)
    wsem = (ws0, ws1)

    def tok0(i):
      return base + i * CHUNK

    def fire_idx(i, b):
      pltpu.async_copy(day_hbm.at[pl.ds(tok0(i), CHUNK)], d_v.at[b], isem[b])
      pltpu.async_copy(time_hbm.at[pl.ds(tok0(i), CHUNK)], t_v.at[b], isem[b])

    def wait_idx(i, b):
      pltpu.make_async_copy(
          day_hbm.at[pl.ds(tok0(i), CHUNK)], d_v.at[b], isem[b]).wait()
      pltpu.make_async_copy(
          time_hbm.at[pl.ds(tok0(i), CHUNK)], t_v.at[b], isem[b]).wait()

    def compute_p(b):
      for g in range(GROUPS):
        d = d_v[b, pl.ds(g * LANES, LANES)]
        t = t_v[b, pl.ds(g * LANES, LANES)]
        d = jnp.clip(d, 0, NUM_DAY - 1)
        t = jnp.clip(t, 0, NUM_TIME - 1)
        p = t * NUM_DAY + d
        gpr = GROW // LANES
        p_v[b, g // gpr, pl.ds((g % gpr) * LANES, LANES)] = p

    def fire_gathers(b):
      for j in range(NGATHER):
        pltpu.async_copy(
            fused_hbm.at[p_v.at[b, j]],
            blk[b].at[pl.ds(j * GROW, GROW)], gsem[b])

    def wait_gathers(b):
      for j in range(NGATHER):
        pltpu.make_async_copy(
            fused_hbm.at[p_v.at[b, j]],
            blk[b].at[pl.ds(j * GROW, GROW)], gsem[b]).wait()

    def fire_wb(i, b):
      pltpu.async_copy(blk[b], out_hbm.at[pl.ds(tok0(i), CHUNK)], wsem[b])

    def wait_wb(i, b):
      pltpu.make_async_copy(
          blk[b], out_hbm.at[pl.ds(tok0(i), CHUNK)], wsem[b]).wait()

    fire_idx(0, 0)
    wait_idx(0, 0)
    compute_p(0)
    fire_idx(1, 1)
    fire_gathers(0)

    @pl.loop(0, iters, step=2)
    def _(o):
      for b in (0, 1):
        i = o + b
        nb = 1 - b
        wait_gathers(b)
        fire_wb(i, b)

        @pl.when(i + 1 < iters)
        def _():
          wait_idx(i + 1, nb)
          compute_p(nb)

          @pl.when(i + 2 < iters)
          def _():
            fire_idx(i + 2, b)

          @pl.when(i > 0)
          def _():
            wait_wb(i - 1, nb)
          fire_gathers(nb)

        @pl.when(jnp.logical_and(i + 1 >= iters, i > 0))
        def _():
          wait_wb(i - 1, nb)

    wait_wb(iters - 1, (iters - 1) % 2)

  return k


def kernel(data_cat, table_day, table_time):
  B, T, _ = data_cat.shape
  n = B * T
  data_cat = data_cat.astype(jnp.int32)
  day_idx = data_cat[:, :, 0].reshape(n)
  time_idx = data_cat[:, :, 1].reshape(n)
  f_time = jnp.broadcast_to(
      table_time[:, None, :], (NUM_TIME, NUM_DAY, TIME_SIZE)
  ).reshape(NUM_FUSED, TIME_SIZE)
  f_day = jnp.broadcast_to(
      table_day[None, :, :], (NUM_TIME, NUM_DAY, DAY_SIZE)
  ).reshape(NUM_FUSED, DAY_SIZE)
  f_pad = jnp.zeros((NUM_FUSED, FPAD - OUT_SIZE), jnp.float32)
  fused = jnp.concatenate([f_time, f_day, f_pad], axis=1)
  out = _sc_embed(n)(day_idx, time_idx, fused)
  return out[:, :OUT_SIZE].reshape(B, T, OUT_SIZE)
